# baseline (speedup 1.0000x reference)
import functools

import jax
import jax.numpy as jnp
from jax.experimental import pallas as pl


def _chamfer_body(x1_ref, x2t_ref, d1_ref, d2_ref, *, num_mb):
    mb = pl.program_id(1)
    a = x1_ref[0]
    bt = x2t_ref[0]
    d = jax.lax.dot_general(
        a, bt, (((1,), (0,)), ((), ())),
        preferred_element_type=jnp.float32,
        precision=jax.lax.Precision.HIGHEST,
    )
    part1 = jnp.min(d, axis=1)

    @pl.when(mb == 0)
    def _():
        d1_ref[0, 0] = part1

    @pl.when(mb > 0)
    def _():
        d1_ref[0, 0] = jnp.minimum(d1_ref[0, 0], part1)

    d2_ref[0, 0] = jnp.min(d, axis=0)


def _chamfer_dists(xyz1, xyz2, *, tm=512, interpret=False):
    B, N, _ = xyz1.shape
    M = xyz2.shape[1]
    num_mb = M // tm
    ones_n = jnp.ones((B, N, 1), jnp.float32)
    x1sq = jnp.sum(xyz1 * xyz1, axis=2, keepdims=True)
    a = jnp.concatenate([-2.0 * xyz1, ones_n, x1sq], axis=2)
    x2t = jnp.transpose(xyz2, (0, 2, 1))
    x2sq = jnp.sum(x2t * x2t, axis=1, keepdims=True)
    ones_m = jnp.ones((B, 1, M), jnp.float32)
    bt = jnp.concatenate([x2t, x2sq, ones_m], axis=1)

    d1, d2 = pl.pallas_call(
        functools.partial(_chamfer_body, num_mb=num_mb),
        grid=(B, num_mb),
        in_specs=[
            pl.BlockSpec((1, N, 5), lambda b, mb: (b, 0, 0)),
            pl.BlockSpec((1, 5, tm), lambda b, mb: (b, 0, mb)),
        ],
        out_specs=[
            pl.BlockSpec((1, 1, N), lambda b, mb: (b, 0, 0)),
            pl.BlockSpec((1, 1, tm), lambda b, mb: (b, 0, mb)),
        ],
        out_shape=[
            jax.ShapeDtypeStruct((B, 1, N), jnp.float32),
            jax.ShapeDtypeStruct((B, 1, M), jnp.float32),
        ],
        interpret=interpret,
    )(a, bt)
    return d1[:, 0, :], d2[:, 0, :]


@jax.jit
def kernel(xyz1, xyz2, weights1, weights2):
    dist1, dist2 = _chamfer_dists(xyz1, xyz2)
    dist1_avg = jnp.sum(dist1 * weights1) / jnp.sum(weights1)
    dist2_avg = jnp.sum(dist2 * weights2) / jnp.sum(weights2)
    return (dist1_avg + dist2_avg) / 2.0


# speedup vs baseline: 2.2874x; 2.2874x over previous
import functools

import jax
import jax.numpy as jnp
from jax.experimental import pallas as pl


def _chamfer_body(x1_ref, x2t_ref, d1_ref, d2_ref, *, num_mb):
    mb = pl.program_id(1)
    a = x1_ref[0]
    bt = x2t_ref[0]
    xm2 = a[:, 0:3]
    x1sq = a[:, 3:4]
    x2 = bt[0:3, :]
    x2sq = bt[3:4, :]
    xyn = jax.lax.dot_general(
        xm2, x2, (((1,), (0,)), ((), ())),
        preferred_element_type=jnp.float32,
    )
    e = xyn + x1sq
    f = xyn + x2sq
    part1 = jnp.min(f, axis=1) + x1sq[:, 0]

    @pl.when(mb == 0)
    def _():
        d1_ref[0, 0] = part1

    @pl.when(mb > 0)
    def _():
        d1_ref[0, 0] = jnp.minimum(d1_ref[0, 0], part1)

    d2_ref[0, 0] = jnp.min(e, axis=0) + x2sq[0]


def _chamfer_dists(xyz1, xyz2, *, tm=512, interpret=False):
    B, N, _ = xyz1.shape
    M = xyz2.shape[1]
    num_mb = M // tm
    x1sq = jnp.sum(xyz1 * xyz1, axis=2, keepdims=True)
    a = jnp.concatenate([-2.0 * xyz1, x1sq], axis=2)
    x2t = jnp.transpose(xyz2, (0, 2, 1))
    x2sq = jnp.sum(x2t * x2t, axis=1, keepdims=True)
    bt = jnp.concatenate([x2t, x2sq], axis=1)

    d1, d2 = pl.pallas_call(
        functools.partial(_chamfer_body, num_mb=num_mb),
        grid=(B, num_mb),
        in_specs=[
            pl.BlockSpec((1, N, 4), lambda b, mb: (b, 0, 0)),
            pl.BlockSpec((1, 4, tm), lambda b, mb: (b, 0, mb)),
        ],
        out_specs=[
            pl.BlockSpec((1, 1, N), lambda b, mb: (b, 0, 0)),
            pl.BlockSpec((1, 1, tm), lambda b, mb: (b, 0, mb)),
        ],
        out_shape=[
            jax.ShapeDtypeStruct((B, 1, N), jnp.float32),
            jax.ShapeDtypeStruct((B, 1, M), jnp.float32),
        ],
        interpret=interpret,
    )(a, bt)
    return d1[:, 0, :], d2[:, 0, :]


@jax.jit
def kernel(xyz1, xyz2, weights1, weights2):
    dist1, dist2 = _chamfer_dists(xyz1, xyz2)
    dist1_avg = jnp.sum(dist1 * weights1) / jnp.sum(weights1)
    dist2_avg = jnp.sum(dist2 * weights2) / jnp.sum(weights2)
    return (dist1_avg + dist2_avg) / 2.0


# speedup vs baseline: 4.3068x; 1.8829x over previous
import functools

import jax
import jax.numpy as jnp
from jax.experimental import pallas as pl
from jax.experimental.pallas import tpu as pltpu


def _chamfer_body(x1_ref, x2t_ref, d1_ref, d2_ref, acc_ref, *, num_mb, tm):
    mb = pl.program_id(1)
    a = x1_ref[0]
    bt = x2t_ref[0]
    xm2 = a[:, 0:3]
    x1sq = a[:, 3:4]
    x2 = bt[0:3, :]
    x2sq = bt[3:4, :]
    xyn = jax.lax.dot_general(
        xm2, x2, (((1,), (0,)), ((), ())),
        preferred_element_type=jnp.float32,
    )

    e = xyn + x1sq
    d2_ref[0, 0] = jnp.min(e, axis=0) + x2sq[0]

    f = xyn + x2sq
    g = f[:, 0:128]
    for k in range(1, tm // 128):
        g = jnp.minimum(g, f[:, k * 128:(k + 1) * 128])

    @pl.when(mb == 0)
    def _():
        acc_ref[...] = g

    @pl.when(mb > 0)
    def _():
        acc_ref[...] = jnp.minimum(acc_ref[...], g)

    @pl.when(mb == num_mb - 1)
    def _():
        d1_ref[0, 0] = jnp.min(acc_ref[...], axis=1) + x1sq[:, 0]


def _chamfer_dists(xyz1, xyz2, *, tm=512, interpret=False):
    B, N, _ = xyz1.shape
    M = xyz2.shape[1]
    num_mb = M // tm
    x1sq = jnp.sum(xyz1 * xyz1, axis=2, keepdims=True)
    a = jnp.concatenate([-2.0 * xyz1, x1sq], axis=2)
    x2t = jnp.transpose(xyz2, (0, 2, 1))
    x2sq = jnp.sum(x2t * x2t, axis=1, keepdims=True)
    bt = jnp.concatenate([x2t, x2sq], axis=1)

    d1, d2 = pl.pallas_call(
        functools.partial(_chamfer_body, num_mb=num_mb, tm=tm),
        grid=(B, num_mb),
        in_specs=[
            pl.BlockSpec((1, N, 4), lambda b, mb: (b, 0, 0)),
            pl.BlockSpec((1, 4, tm), lambda b, mb: (b, 0, mb)),
        ],
        out_specs=[
            pl.BlockSpec((1, 1, N), lambda b, mb: (b, 0, 0)),
            pl.BlockSpec((1, 1, tm), lambda b, mb: (b, 0, mb)),
        ],
        out_shape=[
            jax.ShapeDtypeStruct((B, 1, N), jnp.float32),
            jax.ShapeDtypeStruct((B, 1, M), jnp.float32),
        ],
        scratch_shapes=[pltpu.VMEM((N, 128), jnp.float32)],
        interpret=interpret,
    )(a, bt)
    return d1[:, 0, :], d2[:, 0, :]


@jax.jit
def kernel(xyz1, xyz2, weights1, weights2):
    dist1, dist2 = _chamfer_dists(xyz1, xyz2)
    dist1_avg = jnp.sum(dist1 * weights1) / jnp.sum(weights1)
    dist2_avg = jnp.sum(dist2 * weights2) / jnp.sum(weights2)
    return (dist1_avg + dist2_avg) / 2.0
